# Initial kernel scaffold; baseline (speedup 1.0000x reference)
import jax
import jax.numpy as jnp
from jax.experimental import pallas as pl


def kernel(relation_path, path_info, graph_feature, context_feature, dis_embed, dis_sent_embed):
    raise NotImplementedError("write your pallas kernel here")


# speedup vs baseline: 3.5916x; 3.5916x over previous
import functools

import numpy as np
import jax
import jax.numpy as jnp
from jax import lax
from jax.experimental import pallas as pl
from jax.experimental.pallas import tpu as pltpu
from jax.experimental.pallas import tpu_sc as plsc

_NB = 4
_NP = 1722
_NM = 12
_HID = 552
_RPB = _NP * _NM
_ROWS = _NB * _RPB
_TPB = 8
_CHUNK = 2592
_GROUPS = _CHUNK // 16


def _dis2idx_np():
    d = np.zeros(512, np.int32)
    d[1] = 1
    d[2:] = 2
    d[4:] = 3
    d[8:] = 4
    d[16:] = 5
    d[32:] = 6
    d[64:] = 7
    d[128:] = 8
    d[256:] = 9
    return d


_DIS2IDX = _dis2idx_np()


def _make_sc_call():
    mesh = plsc.VectorSubcoreMesh(core_axis_name="c", subcore_axis_name="s")

    @functools.partial(
        pl.kernel,
        mesh=mesh,
        compiler_params=pltpu.CompilerParams(needs_layout_passes=False),
        out_type=[
            jax.ShapeDtypeStruct((_ROWS, _HID), jnp.float32),
            jax.ShapeDtypeStruct((_ROWS,), jnp.int32),
        ],
        scratch_types=[
            pltpu.VMEM((256, 128), jnp.float32),
            pltpu.VMEM((512, 128), jnp.float32),
            pltpu.VMEM((256,), jnp.int32),
            pltpu.VMEM((40, 20), jnp.float32),
            pltpu.VMEM((512,), jnp.int32),
            pltpu.VMEM((64,), jnp.int32),
            pltpu.VMEM((16, _HID), jnp.float32),
            pltpu.VMEM((_CHUNK,), jnp.int32),
        ],
    )
    def sc_kernel(rel, pinfo, graph, ctx, discat, d2i, outf, outm,
                  graph_v, ctx_v, pinfo_v, discat_v, d2i_v, idx_v,
                  stage_v, mask_v):
        cid = lax.axis_index("c")
        sid = lax.axis_index("s")
        wid = sid * 2 + cid
        b = wid // _TPB
        t8 = wid % _TPB
        local_base = jnp.minimum(t8 * _CHUNK, _RPB - _CHUNK)
        base = b * _RPB + local_base

        pltpu.sync_copy(graph.at[b], graph_v)
        pltpu.sync_copy(ctx.at[b], ctx_v)
        pltpu.sync_copy(pinfo.at[b], pinfo_v)
        pltpu.sync_copy(discat, discat_v)
        pltpu.sync_copy(d2i, d2i_v)

        lane = lax.iota(jnp.int32, 16)
        lane4 = lane * 4

        def group_body(g, carry):
            start = base + g * 16
            pltpu.sync_copy(rel.at[pl.ds(start * 4, 64)], idx_v)
            i0 = jnp.clip(plsc.load_gather(idx_v, [lane4]), 0, 255)
            i1 = jnp.clip(plsc.load_gather(idx_v, [lane4 + 1]), 0, 255)
            i2 = jnp.clip(plsc.load_gather(idx_v, [lane4 + 2]), 0, 255)
            i3 = jnp.clip(plsc.load_gather(idx_v, [lane4 + 3]), 0, 255)
            ssum = i0 + i1 + i2 + i3
            mask_f = jnp.where(ssum > 0, 1.0, 0.0).astype(jnp.float32)
            plsc.store_scatter(mask_v, [g * 16 + lane], ssum)

            m = lax.rem(local_base + g * 16 + lane, _NM)
            is0 = m < 4
            is2 = m >= 8
            isel = jnp.where(is0, i2, i3)
            wlog = jnp.where(is2, 1.0, 0.0).astype(jnp.float32)

            a0 = jnp.clip(plsc.load_gather(pinfo_v, [i0]), 0, 511)
            a1 = jnp.clip(plsc.load_gather(pinfo_v, [i1]), 0, 511)
            a2 = jnp.clip(plsc.load_gather(pinfo_v, [i2]), 0, 511)
            a3 = jnp.clip(plsc.load_gather(pinfo_v, [i3]), 0, 511)
            asel = jnp.where(is0, a2, a3)

            delta = a0 - asel
            xeff = jnp.clip(jnp.where(delta < 0, delta + 512, delta), 0, 511)
            d = plsc.load_gather(d2i_v, [xeff])
            di = jnp.where(delta < 0, 10 - d, 10 + d)
            di2 = plsc.load_gather(d2i_v, [di]) + 10 + 20

            def dis_body(col, cc):
                cvec = jnp.broadcast_to(col, (16,))
                e0 = plsc.load_gather(discat_v, [di, cvec]) * mask_f
                plsc.store_scatter(stage_v, [lane, cvec + 256], e0)
                e1 = plsc.load_gather(discat_v, [di2, cvec]) * mask_f
                plsc.store_scatter(stage_v, [lane, cvec + 276], e1)
                return cc
            lax.fori_loop(0, 20, dis_body, 0)

            def col_body(col, cc):
                cvec = jnp.broadcast_to(col, (16,))
                v0 = plsc.load_gather(graph_v, [i0, cvec]) * mask_f
                plsc.store_scatter(stage_v, [lane, cvec], v0)
                v1 = plsc.load_gather(graph_v, [isel, cvec]) * mask_f
                plsc.store_scatter(stage_v, [lane, cvec + 128], v1)
                u0 = plsc.load_gather(ctx_v, [a0, cvec])
                u1 = plsc.load_gather(ctx_v, [a1, cvec])
                plsc.store_scatter(stage_v, [lane, cvec + 296],
                                   (u0 + wlog * u1) * mask_f)
                u2 = plsc.load_gather(ctx_v, [a2, cvec])
                u3 = plsc.load_gather(ctx_v, [a3, cvec])
                plsc.store_scatter(stage_v, [lane, cvec + 424],
                                   (u2 + wlog * u3) * mask_f)
                return cc
            lax.fori_loop(0, 128, col_body, 0)

            pltpu.sync_copy(stage_v, outf.at[pl.ds(start, 16)])
            return carry

        lax.fori_loop(0, _GROUPS, group_body, 0)
        pltpu.sync_copy(mask_v, outm.at[pl.ds(base, _CHUNK)])

    return sc_kernel


_sc_call = _make_sc_call()


@jax.jit
def kernel(relation_path, path_info, graph_feature, context_feature,
           dis_embed, dis_sent_embed):
    rel1 = relation_path.astype(jnp.int32).reshape(_ROWS * 4)
    pinfo0 = path_info.astype(jnp.int32)[:, :, 0]
    gf = graph_feature.astype(jnp.float32)
    cf = context_feature.astype(jnp.float32)
    discat = jnp.concatenate(
        [dis_embed.astype(jnp.float32), dis_sent_embed.astype(jnp.float32)],
        axis=0)
    d2i = jnp.asarray(_DIS2IDX)
    outf, outm = _sc_call(rel1, pinfo0, gf, cf, discat, d2i)
    path_fea = outf.reshape(_NB, _NP, _NM, _HID)
    mask = outm.reshape(_NB, _NP, _NM) > 0
    return (path_fea, mask)


# speedup vs baseline: 4.4856x; 1.2489x over previous
import functools

import jax
import jax.numpy as jnp
from jax import lax
from jax.experimental import pallas as pl
from jax.experimental.pallas import tpu as pltpu
from jax.experimental.pallas import tpu_sc as plsc

_NB = 4
_NP = 1722
_NM = 12
_HID = 552
_RPB = _NP * _NM
_ROWS = _NB * _RPB
_TPB = 8
_CHUNK = 2592
_GROUPS = _CHUNK // 16
_GBLK = 16 * _HID


def _bucket(x):
    e = lax.shift_right_logical(plsc.bitcast(x.astype(jnp.float32), jnp.int32), 23)
    return jnp.maximum(e - 126, 0)


def _make_sc_call():
    mesh = plsc.VectorSubcoreMesh(core_axis_name="c", subcore_axis_name="s")

    @functools.partial(
        pl.kernel,
        mesh=mesh,
        compiler_params=pltpu.CompilerParams(needs_layout_passes=False),
        out_type=[
            jax.ShapeDtypeStruct((_ROWS * _HID,), jnp.float32),
            jax.ShapeDtypeStruct((_ROWS,), jnp.int32),
        ],
        scratch_types=[
            pltpu.VMEM((256 * 128,), jnp.float32),
            pltpu.VMEM((512 * 128,), jnp.float32),
            pltpu.VMEM((256,), jnp.int32),
            pltpu.VMEM((800,), jnp.float32),
            pltpu.VMEM((_CHUNK * 4,), jnp.int32),
            pltpu.VMEM((2 * _GBLK,), jnp.float32),
            pltpu.VMEM((_CHUNK,), jnp.int32),
            pltpu.SemaphoreType.DMA((2,)),
        ],
    )
    def sc_kernel(rel, pinfo, graph, ctx, discat, outf, outm,
                  graph_v, ctx_v, pinfo_v, discat_v, idx_v,
                  stage_v, mask_v, sem):
        cid = lax.axis_index("c")
        sid = lax.axis_index("s")
        wid = sid * 2 + cid
        b = wid // _TPB
        t8 = wid % _TPB
        local_base = jnp.minimum(t8 * _CHUNK, _RPB - _CHUNK)
        base = b * _RPB + local_base

        pltpu.sync_copy(graph.at[b], graph_v)
        pltpu.sync_copy(ctx.at[b], ctx_v)
        pltpu.sync_copy(pinfo.at[b], pinfo_v)
        pltpu.sync_copy(discat, discat_v)
        pltpu.sync_copy(rel.at[pl.ds(base * 4, _CHUNK * 4)], idx_v)

        lane = lax.iota(jnp.int32, 16)
        lane4 = lane * 4
        lane552 = lane * _HID

        def group_body(g, carry):
            start = base + g * 16
            par = lax.rem(g, 2)
            sbuf = par * _GBLK

            @pl.when(g >= 2)
            def _wait():
                pltpu.make_async_copy(
                    stage_v.at[pl.ds(sbuf, _GBLK)],
                    outf.at[pl.ds((start - 32) * _HID, _GBLK)],
                    sem.at[par]).wait()

            goff = g * 64 + lane4
            i0 = jnp.clip(plsc.load_gather(idx_v, [goff]), 0, 255)
            i1 = jnp.clip(plsc.load_gather(idx_v, [goff + 1]), 0, 255)
            i2 = jnp.clip(plsc.load_gather(idx_v, [goff + 2]), 0, 255)
            i3 = jnp.clip(plsc.load_gather(idx_v, [goff + 3]), 0, 255)
            ssum = i0 + i1 + i2 + i3
            mask_f = jnp.where(ssum > 0, 1.0, 0.0).astype(jnp.float32)
            plsc.store_scatter(mask_v, [g * 16 + lane], ssum)

            m = lax.rem(local_base + g * 16 + lane, _NM)
            is0 = m < 4
            is2 = m >= 8
            isel = jnp.where(is0, i2, i3)
            wlog = jnp.where(is2, 1.0, 0.0).astype(jnp.float32)

            a0 = jnp.clip(plsc.load_gather(pinfo_v, [i0]), 0, 511)
            a1 = jnp.clip(plsc.load_gather(pinfo_v, [i1]), 0, 511)
            a2 = jnp.clip(plsc.load_gather(pinfo_v, [i2]), 0, 511)
            a3 = jnp.clip(plsc.load_gather(pinfo_v, [i3]), 0, 511)
            asel = jnp.where(is0, a2, a3)

            delta = a0 - asel
            xeff = jnp.clip(jnp.where(delta < 0, delta + 512, delta), 0, 511)
            d = _bucket(xeff)
            di = jnp.where(delta < 0, 10 - d, 10 + d)
            di2 = _bucket(di) + 30

            gA0 = i0 * 128
            gA1 = isel * 128
            cA0 = a0 * 128
            cA1 = a1 * 128
            cA2 = a2 * 128
            cA3 = a3 * 128
            dA0 = di * 20
            dA1 = di2 * 20
            sb = sbuf + lane552

            def dis_body(c4, cc):
                col = c4 * 4
                for u in range(4):
                    cu = col + u
                    e0 = plsc.load_gather(discat_v, [dA0 + cu]) * mask_f
                    plsc.store_scatter(stage_v, [sb + (256 + cu)], e0)
                    e1 = plsc.load_gather(discat_v, [dA1 + cu]) * mask_f
                    plsc.store_scatter(stage_v, [sb + (276 + cu)], e1)
                return cc
            lax.fori_loop(0, 5, dis_body, 0)

            def col_body(c4, cc):
                col = c4 * 4
                for u in range(4):
                    cu = col + u
                    v0 = plsc.load_gather(graph_v, [gA0 + cu]) * mask_f
                    plsc.store_scatter(stage_v, [sb + cu], v0)
                    v1 = plsc.load_gather(graph_v, [gA1 + cu]) * mask_f
                    plsc.store_scatter(stage_v, [sb + (128 + cu)], v1)
                    u0 = plsc.load_gather(ctx_v, [cA0 + cu])
                    u1 = plsc.load_gather(ctx_v, [cA1 + cu])
                    plsc.store_scatter(stage_v, [sb + (296 + cu)],
                                       (u0 + wlog * u1) * mask_f)
                    u2 = plsc.load_gather(ctx_v, [cA2 + cu])
                    u3 = plsc.load_gather(ctx_v, [cA3 + cu])
                    plsc.store_scatter(stage_v, [sb + (424 + cu)],
                                       (u2 + wlog * u3) * mask_f)
                return cc
            lax.fori_loop(0, 32, col_body, 0)

            pltpu.async_copy(
                stage_v.at[pl.ds(sbuf, _GBLK)],
                outf.at[pl.ds(start * _HID, _GBLK)],
                sem.at[par])
            return carry

        lax.fori_loop(0, _GROUPS, group_body, 0)

        last0 = base + (_GROUPS - 2) * 16
        pltpu.make_async_copy(
            stage_v.at[pl.ds(0, _GBLK)],
            outf.at[pl.ds(last0 * _HID, _GBLK)], sem.at[0]).wait()
        pltpu.make_async_copy(
            stage_v.at[pl.ds(_GBLK, _GBLK)],
            outf.at[pl.ds((last0 + 16) * _HID, _GBLK)], sem.at[1]).wait()

        pltpu.sync_copy(mask_v, outm.at[pl.ds(base, _CHUNK)])

    return sc_kernel


_sc_call = _make_sc_call()


@jax.jit
def kernel(relation_path, path_info, graph_feature, context_feature,
           dis_embed, dis_sent_embed):
    rel1 = relation_path.astype(jnp.int32).reshape(_ROWS * 4)
    pinfo0 = path_info.astype(jnp.int32)[:, :, 0]
    gf = graph_feature.astype(jnp.float32).reshape(_NB, 256 * 128)
    cf = context_feature.astype(jnp.float32).reshape(_NB, 512 * 128)
    discat = jnp.concatenate(
        [dis_embed.astype(jnp.float32), dis_sent_embed.astype(jnp.float32)],
        axis=0).reshape(800)
    outf, outm = _sc_call(rel1, pinfo0, gf, cf, discat)
    path_fea = outf.reshape(_NB, _NP, _NM, _HID)
    mask = outm.reshape(_NB, _NP, _NM) > 0
    return (path_fea, mask)


# speedup vs baseline: 5.9421x; 1.3247x over previous
import functools

import jax
import jax.numpy as jnp
from jax import lax
from jax.experimental import pallas as pl
from jax.experimental.pallas import tpu as pltpu
from jax.experimental.pallas import tpu_sc as plsc

_NB = 4
_NP = 1722
_NM = 12
_HID = 552
_RPB = _NP * _NM
_ROWS = _NB * _RPB
_TPB = 8
_CHUNK = 2592
_GROUPS = _CHUNK // 16
_GBLK = 16 * _HID


def _bucket(x):
    e = lax.shift_right_logical(plsc.bitcast(x.astype(jnp.float32), jnp.int32), 23)
    return jnp.maximum(e - 126, 0)


def _make_sc_call():
    mesh = plsc.VectorSubcoreMesh(core_axis_name="c", subcore_axis_name="s")

    @functools.partial(
        pl.kernel,
        mesh=mesh,
        compiler_params=pltpu.CompilerParams(needs_layout_passes=False),
        out_type=[
            jax.ShapeDtypeStruct((_ROWS * _HID,), jnp.float32),
            jax.ShapeDtypeStruct((_ROWS,), jnp.int32),
        ],
        scratch_types=[
            pltpu.VMEM((256 * 128,), jnp.float32),
            pltpu.VMEM((512 * 128,), jnp.float32),
            pltpu.VMEM((256,), jnp.int32),
            pltpu.VMEM((800,), jnp.float32),
            pltpu.VMEM((_CHUNK * 4,), jnp.int32),
            pltpu.VMEM((2 * _GBLK,), jnp.float32),
            pltpu.VMEM((_CHUNK,), jnp.int32),
            pltpu.SemaphoreType.DMA((2,)),
        ],
    )
    def sc_kernel(rel, pinfo, graph, ctx, discat, outf, outm,
                  graph_v, ctx_v, pinfo_v, discat_v, idx_v,
                  stage_v, mask_v, sem):
        cid = lax.axis_index("c")
        sid = lax.axis_index("s")
        wid = sid * 2 + cid
        b = wid // _TPB
        t8 = wid % _TPB
        local_base = jnp.minimum(t8 * _CHUNK, _RPB - _CHUNK)
        base = b * _RPB + local_base

        pltpu.sync_copy(graph.at[b], graph_v)
        pltpu.sync_copy(ctx.at[b], ctx_v)
        pltpu.sync_copy(pinfo.at[b], pinfo_v)
        pltpu.sync_copy(discat, discat_v)
        pltpu.sync_copy(rel.at[pl.ds(base * 4, _CHUNK * 4)], idx_v)

        lane = lax.iota(jnp.int32, 16)
        lane4 = lane * 4
        lane552 = lane * _HID

        def group_body(g, carry):
            start = base + g * 16
            par = lax.rem(g, 2)
            sbuf = par * _GBLK

            @pl.when(g >= 2)
            def _wait():
                pltpu.make_async_copy(
                    stage_v.at[pl.ds(sbuf, _GBLK)],
                    outf.at[pl.ds((start - 32) * _HID, _GBLK)],
                    sem.at[par]).wait()

            goff = g * 64 + lane4
            i0 = jnp.clip(plsc.load_gather(idx_v, [goff]), 0, 255)
            i1 = jnp.clip(plsc.load_gather(idx_v, [goff + 1]), 0, 255)
            i2 = jnp.clip(plsc.load_gather(idx_v, [goff + 2]), 0, 255)
            i3 = jnp.clip(plsc.load_gather(idx_v, [goff + 3]), 0, 255)
            ssum = i0 + i1 + i2 + i3
            mask_f = jnp.where(ssum > 0, 1.0, 0.0).astype(jnp.float32)
            plsc.store_scatter(mask_v, [g * 16 + lane], ssum)

            m = lax.rem(local_base + g * 16 + lane, _NM)
            is0 = m < 4
            is2 = m >= 8
            isel = jnp.where(is0, i2, i3)
            wlog = jnp.where(is2, 1.0, 0.0).astype(jnp.float32)

            a0 = jnp.clip(plsc.load_gather(pinfo_v, [i0]), 0, 511)
            a1 = jnp.clip(plsc.load_gather(pinfo_v, [i1]), 0, 511)
            a2 = jnp.clip(plsc.load_gather(pinfo_v, [i2]), 0, 511)
            a3 = jnp.clip(plsc.load_gather(pinfo_v, [i3]), 0, 511)
            asel = jnp.where(is0, a2, a3)

            delta = a0 - asel
            xeff = jnp.clip(jnp.where(delta < 0, delta + 512, delta), 0, 511)
            d = _bucket(xeff)
            di = jnp.where(delta < 0, 10 - d, 10 + d)
            di2 = _bucket(di) + 30

            gA0 = i0 * 128
            gA1 = isel * 128
            cA0 = a0 * 128
            cA1 = a1 * 128
            cA2 = a2 * 128
            cA3 = a3 * 128
            dA0 = di * 20
            dA1 = di2 * 20
            sb = sbuf + lane552

            @plsc.parallel_loop(0, 20, unroll=4)
            def _dis_body(cu):
                e0 = plsc.load_gather(discat_v, [dA0 + cu]) * mask_f
                plsc.store_scatter(stage_v, [sb + (256 + cu)], e0)
                e1 = plsc.load_gather(discat_v, [dA1 + cu]) * mask_f
                plsc.store_scatter(stage_v, [sb + (276 + cu)], e1)

            @plsc.parallel_loop(0, 128, unroll=4)
            def _col_body(cu):
                v0 = plsc.load_gather(graph_v, [gA0 + cu]) * mask_f
                plsc.store_scatter(stage_v, [sb + cu], v0)
                v1 = plsc.load_gather(graph_v, [gA1 + cu]) * mask_f
                plsc.store_scatter(stage_v, [sb + (128 + cu)], v1)
                u0 = plsc.load_gather(ctx_v, [cA0 + cu])
                u1 = plsc.load_gather(ctx_v, [cA1 + cu])
                plsc.store_scatter(stage_v, [sb + (296 + cu)],
                                   (u0 + wlog * u1) * mask_f)
                u2 = plsc.load_gather(ctx_v, [cA2 + cu])
                u3 = plsc.load_gather(ctx_v, [cA3 + cu])
                plsc.store_scatter(stage_v, [sb + (424 + cu)],
                                   (u2 + wlog * u3) * mask_f)

            pltpu.async_copy(
                stage_v.at[pl.ds(sbuf, _GBLK)],
                outf.at[pl.ds(start * _HID, _GBLK)],
                sem.at[par])
            return carry

        lax.fori_loop(0, _GROUPS, group_body, 0)

        last0 = base + (_GROUPS - 2) * 16
        pltpu.make_async_copy(
            stage_v.at[pl.ds(0, _GBLK)],
            outf.at[pl.ds(last0 * _HID, _GBLK)], sem.at[0]).wait()
        pltpu.make_async_copy(
            stage_v.at[pl.ds(_GBLK, _GBLK)],
            outf.at[pl.ds((last0 + 16) * _HID, _GBLK)], sem.at[1]).wait()

        pltpu.sync_copy(mask_v, outm.at[pl.ds(base, _CHUNK)])

    return sc_kernel


_sc_call = _make_sc_call()


@jax.jit
def kernel(relation_path, path_info, graph_feature, context_feature,
           dis_embed, dis_sent_embed):
    rel1 = relation_path.astype(jnp.int32).reshape(_ROWS * 4)
    pinfo0 = path_info.astype(jnp.int32)[:, :, 0]
    gf = graph_feature.astype(jnp.float32).reshape(_NB, 256 * 128)
    cf = context_feature.astype(jnp.float32).reshape(_NB, 512 * 128)
    discat = jnp.concatenate(
        [dis_embed.astype(jnp.float32), dis_sent_embed.astype(jnp.float32)],
        axis=0).reshape(800)
    outf, outm = _sc_call(rel1, pinfo0, gf, cf, discat)
    path_fea = outf.reshape(_NB, _NP, _NM, _HID)
    mask = outm.reshape(_NB, _NP, _NM) > 0
    return (path_fea, mask)


# speedup vs baseline: 8.3544x; 1.4060x over previous
import functools

import jax
import jax.numpy as jnp
from jax import lax
from jax.experimental import pallas as pl
from jax.experimental.pallas import tpu as pltpu
from jax.experimental.pallas import tpu_sc as plsc

_NB = 4
_NP = 1722
_NM = 12
_HID = 552
_NPP = 1792
_NK = 14
_JOBS = _NB * _NM * _NK
_NA = 12
_NBT = 20
_AJOBS = _JOBS // _NA
_BJOBS = -(-_JOBS // _NBT)
_JPB = _NM * _NK
_RPAD = (_NPP - _NP) * 48


def _bucket(x):
    e = lax.shift_right_logical(plsc.bitcast(x.astype(jnp.float32), jnp.int32), 23)
    return jnp.maximum(e - 126, 0)


def _make_sc_call():
    mesh = plsc.VectorSubcoreMesh(core_axis_name="c", subcore_axis_name="s")

    @functools.partial(
        pl.kernel,
        mesh=mesh,
        compiler_params=pltpu.CompilerParams(needs_layout_passes=False),
        out_type=[
            jax.ShapeDtypeStruct((_NB, _NM, _HID, _NPP), jnp.float32),
            jax.ShapeDtypeStruct((_NA, _AJOBS * 128), jnp.int32),
        ],
        scratch_types=[
            pltpu.VMEM((500, 128), jnp.float32),
            pltpu.VMEM((40, 20), jnp.float32),
            pltpu.VMEM((256,), jnp.int32),
            pltpu.VMEM((768,), jnp.int32),
            pltpu.VMEM((144, 128), jnp.float32),
            pltpu.VMEM((152, 128), jnp.float32),
            pltpu.VMEM((_AJOBS * 128,), jnp.int32),
            pltpu.SemaphoreType.DMA((2,)),
        ],
    )
    def sc_kernel(rel, pinfo, graph, ctx, discat, outf, outm,
                  table_v, discat_v, pinfo_v, idx_v, bufx_v, bufy_v,
                  mask_v, sem):
        cid = lax.axis_index("c")
        sid = lax.axis_index("s")
        wid = sid * 2 + cid
        lane = lax.iota(jnp.int32, 16)
        lane48 = lane * 48

        def load_ids(b, m, pq):
            pltpu.sync_copy(rel.at[pl.ds((b * _NP + pq) * 48, 768)], idx_v)
            koff = lane48 + m * 4
            i0 = jnp.clip(plsc.load_gather(idx_v, [koff]), 0, 255)
            i1 = jnp.clip(plsc.load_gather(idx_v, [koff + 1]), 0, 255)
            i2 = jnp.clip(plsc.load_gather(idx_v, [koff + 2]), 0, 255)
            i3 = jnp.clip(plsc.load_gather(idx_v, [koff + 3]), 0, 255)
            return i0, i1, i2, i3

        @pl.when(wid < _NA)
        def _a_role():
            ta = wid
            b0 = ta // (_NA // _NB)
            pltpu.sync_copy(graph.at[b0], table_v.at[pl.ds(0, 256)])
            pltpu.sync_copy(discat, discat_v)
            pltpu.sync_copy(pinfo.at[b0], pinfo_v)

            def unit(u, carry):
                i = u // 2
                half = lax.rem(u, 2)
                r = lax.rem(ta * _AJOBS + i, _JPB)
                m = r // _NK
                k = lax.rem(r, _NK)

                @pl.when(jnp.logical_and(u >= 2, half == 0))
                def _wx():
                    pltpu.make_async_copy(
                        bufx_v,
                        outf.at[b0, 0, pl.ds(0, 144), pl.ds(0, 128)],
                        sem.at[0]).wait()

                @pl.when(jnp.logical_and(u >= 2, half == 1))
                def _wy():
                    pltpu.make_async_copy(
                        bufy_v,
                        outf.at[b0, 0, pl.ds(144, 152), pl.ds(0, 128)],
                        sem.at[1]).wait()

                def subchunk(c, cc):
                    pq = k * 128 + c * 16
                    i0, i1, i2, i3 = load_ids(b0, m, pq)
                    isel = jnp.where(jnp.broadcast_to(m < 4, (16,)), i2, i3)
                    ssum = i0 + i1 + i2 + i3
                    mask_f = jnp.where(ssum > 0, 1.0, 0.0).astype(jnp.float32)
                    cl = c * 16 + lane

                    @pl.when(half == 0)
                    def _h0():
                        plsc.store_scatter(mask_v, [i * 128 + cl], ssum)

                        @plsc.parallel_loop(0, 128, unroll=4)
                        def _g0(cu):
                            cuv = jnp.broadcast_to(cu, (16,))
                            v = plsc.load_gather(table_v, [i0, cuv]) * mask_f
                            plsc.store_scatter(bufx_v, [cuv, cl], v)

                        @plsc.parallel_loop(0, 16, unroll=4)
                        def _g1(cu):
                            cuv = jnp.broadcast_to(cu, (16,))
                            v = plsc.load_gather(table_v, [isel, cuv]) * mask_f
                            plsc.store_scatter(bufx_v, [cuv + 128, cl], v)

                    @pl.when(half == 1)
                    def _h1():
                        a0 = plsc.load_gather(pinfo_v, [i0])
                        asel = plsc.load_gather(pinfo_v, [isel])
                        delta = a0 - asel
                        xeff = jnp.clip(
                            jnp.where(delta < 0, delta + 512, delta), 0, 511)
                        d = _bucket(xeff)
                        di = jnp.where(delta < 0, 10 - d, 10 + d)
                        di2 = _bucket(di) + 30

                        @plsc.parallel_loop(0, 112, unroll=4)
                        def _g2(cu):
                            cuv = jnp.broadcast_to(cu, (16,))
                            v = plsc.load_gather(
                                table_v, [isel, cuv + 16]) * mask_f
                            plsc.store_scatter(bufy_v, [cuv, cl], v)

                        @plsc.parallel_loop(0, 20, unroll=4)
                        def _g3(cu):
                            cuv = jnp.broadcast_to(cu, (16,))
                            e0 = plsc.load_gather(discat_v, [di, cuv]) * mask_f
                            plsc.store_scatter(bufy_v, [cuv + 112, cl], e0)
                            e1 = plsc.load_gather(discat_v, [di2, cuv]) * mask_f
                            plsc.store_scatter(bufy_v, [cuv + 132, cl], e1)

                    return cc
                lax.fori_loop(0, 8, subchunk, 0)

                @pl.when(half == 0)
                def _dx():
                    pltpu.async_copy(
                        bufx_v,
                        outf.at[b0, m, pl.ds(0, 144), pl.ds(k * 128, 128)],
                        sem.at[0])

                @pl.when(half == 1)
                def _dy():
                    pltpu.async_copy(
                        bufy_v,
                        outf.at[b0, m, pl.ds(144, 152), pl.ds(k * 128, 128)],
                        sem.at[1])
                return carry

            lax.fori_loop(0, 2 * _AJOBS, unit, 0)
            pltpu.make_async_copy(
                bufx_v, outf.at[b0, 0, pl.ds(0, 144), pl.ds(0, 128)],
                sem.at[0]).wait()
            pltpu.make_async_copy(
                bufy_v, outf.at[b0, 0, pl.ds(144, 152), pl.ds(0, 128)],
                sem.at[1]).wait()
            pltpu.sync_copy(mask_v, outm.at[ta])

        @pl.when(wid >= _NA)
        def _b_role():
            tb = wid - _NA
            b0 = (tb * _BJOBS) // _JPB
            pltpu.sync_copy(ctx.at[b0], table_v)
            pltpu.sync_copy(pinfo.at[b0], pinfo_v)

            def unit(u, bcur):
                i = u // 2
                half = lax.rem(u, 2)
                j = jnp.minimum(tb * _BJOBS + i, _JOBS - 1)
                bj = j // _JPB
                r = lax.rem(j, _JPB)
                m = r // _NK
                k = lax.rem(r, _NK)

                @pl.when(bj != bcur)
                def _reload():
                    pltpu.sync_copy(ctx.at[bj], table_v)
                    pltpu.sync_copy(pinfo.at[bj], pinfo_v)

                @pl.when(jnp.logical_and(u >= 2, half == 0))
                def _wx():
                    pltpu.make_async_copy(
                        bufx_v.at[pl.ds(0, 128)],
                        outf.at[bj, 0, pl.ds(296, 128), pl.ds(0, 128)],
                        sem.at[0]).wait()

                @pl.when(jnp.logical_and(u >= 2, half == 1))
                def _wy():
                    pltpu.make_async_copy(
                        bufy_v.at[pl.ds(0, 128)],
                        outf.at[bj, 0, pl.ds(424, 128), pl.ds(0, 128)],
                        sem.at[1]).wait()

                def subchunk(c, cc):
                    pq = k * 128 + c * 16
                    i0, i1, i2, i3 = load_ids(bj, m, pq)
                    ssum = i0 + i1 + i2 + i3
                    mask_f = jnp.where(ssum > 0, 1.0, 0.0).astype(jnp.float32)
                    wlog = jnp.where(jnp.broadcast_to(m >= 8, (16,)),
                                     1.0, 0.0).astype(jnp.float32)
                    a0 = jnp.clip(plsc.load_gather(pinfo_v, [i0]), 0, 499)
                    a1 = jnp.clip(plsc.load_gather(pinfo_v, [i1]), 0, 499)
                    a2 = jnp.clip(plsc.load_gather(pinfo_v, [i2]), 0, 499)
                    a3 = jnp.clip(plsc.load_gather(pinfo_v, [i3]), 0, 499)
                    cl = c * 16 + lane

                    @pl.when(half == 0)
                    def _h0():
                        @plsc.parallel_loop(0, 128, unroll=4)
                        def _c0(cu):
                            cuv = jnp.broadcast_to(cu, (16,))
                            u0 = plsc.load_gather(table_v, [a0, cuv])
                            u1 = plsc.load_gather(table_v, [a1, cuv])
                            plsc.store_scatter(bufx_v, [cuv, cl],
                                               (u0 + wlog * u1) * mask_f)

                    @pl.when(half == 1)
                    def _h1():
                        @plsc.parallel_loop(0, 128, unroll=4)
                        def _c1(cu):
                            cuv = jnp.broadcast_to(cu, (16,))
                            u2 = plsc.load_gather(table_v, [a2, cuv])
                            u3 = plsc.load_gather(table_v, [a3, cuv])
                            plsc.store_scatter(bufy_v, [cuv, cl],
                                               (u2 + wlog * u3) * mask_f)

                    return cc
                lax.fori_loop(0, 8, subchunk, 0)

                @pl.when(half == 0)
                def _dx():
                    pltpu.async_copy(
                        bufx_v.at[pl.ds(0, 128)],
                        outf.at[bj, m, pl.ds(296, 128), pl.ds(k * 128, 128)],
                        sem.at[0])

                @pl.when(half == 1)
                def _dy():
                    pltpu.async_copy(
                        bufy_v.at[pl.ds(0, 128)],
                        outf.at[bj, m, pl.ds(424, 128), pl.ds(k * 128, 128)],
                        sem.at[1])
                return bj

            lax.fori_loop(0, 2 * _BJOBS, unit, b0)
            pltpu.make_async_copy(
                bufx_v.at[pl.ds(0, 128)],
                outf.at[0, 0, pl.ds(296, 128), pl.ds(0, 128)],
                sem.at[0]).wait()
            pltpu.make_async_copy(
                bufy_v.at[pl.ds(0, 128)],
                outf.at[0, 0, pl.ds(424, 128), pl.ds(0, 128)],
                sem.at[1]).wait()

    return sc_kernel


_sc_call = _make_sc_call()


@jax.jit
def kernel(relation_path, path_info, graph_feature, context_feature,
           dis_embed, dis_sent_embed):
    rel1 = jnp.concatenate([
        relation_path.astype(jnp.int32).reshape(_NB * _NP * _NM * 4),
        jnp.zeros((_RPAD,), jnp.int32)])
    pinfo0 = path_info.astype(jnp.int32)[:, :, 0]
    gf = graph_feature.astype(jnp.float32)
    cf = context_feature.astype(jnp.float32)[:, :500, :]
    discat = jnp.concatenate(
        [dis_embed.astype(jnp.float32), dis_sent_embed.astype(jnp.float32)],
        axis=0)
    outf, outm = _sc_call(rel1, pinfo0, gf, cf, discat)
    path_fea = jnp.transpose(outf, (0, 3, 1, 2))[:, :_NP]
    mm = outm.reshape(_JOBS, 128).reshape(_NB, _NM, _NK * 128)[:, :, :_NP]
    mask = jnp.transpose(mm > 0, (0, 2, 1))
    return (path_fea, mask)


# speedup vs baseline: 12.3732x; 1.4810x over previous
import functools

import jax
import jax.numpy as jnp
from jax import lax
from jax.experimental import pallas as pl
from jax.experimental.pallas import tpu as pltpu
from jax.experimental.pallas import tpu_sc as plsc

_NB = 4
_NP = 1722
_NM = 12
_HID = 552
_NPP = 1792
_NK = 14
_JOBS = _NB * _NM * _NK
_NA = 12
_NBT = 20
_AJOBS = _JOBS // _NA
_BJOBS = -(-_JOBS // _NBT)
_JPB = _NM * _NK
_RPAD = (_NPP - _NP) * 48


def _bucket(x):
    e = lax.shift_right_logical(plsc.bitcast(x.astype(jnp.float32), jnp.int32), 23)
    return jnp.maximum(e - 126, 0)


def _make_sc_call():
    mesh = plsc.VectorSubcoreMesh(core_axis_name="c", subcore_axis_name="s")

    @functools.partial(
        pl.kernel,
        mesh=mesh,
        compiler_params=pltpu.CompilerParams(needs_layout_passes=False),
        out_type=[
            jax.ShapeDtypeStruct((_NB, _NM, _HID, _NPP), jnp.float32),
            jax.ShapeDtypeStruct((_NA, _AJOBS * 128), jnp.int32),
        ],
        scratch_types=[
            pltpu.VMEM((500, 128), jnp.float32),
            pltpu.VMEM((40, 20), jnp.float32),
            pltpu.VMEM((256,), jnp.int32),
            pltpu.VMEM((2 * 6144,), jnp.int32),
            pltpu.VMEM((144, 128), jnp.float32),
            pltpu.VMEM((152, 128), jnp.float32),
            pltpu.VMEM((_AJOBS * 128,), jnp.int32),
            pltpu.SemaphoreType.DMA((2,)),
            pltpu.SemaphoreType.DMA,
        ],
    )
    def sc_kernel(rel, pinfo, graph, ctx, discat, outf, outm,
                  table_v, discat_v, pinfo_v, idx_v, bufx_v, bufy_v,
                  mask_v, sem, sem2):
        cid = lax.axis_index("c")
        sid = lax.axis_index("s")
        wid = sid * 2 + cid
        lane = lax.iota(jnp.int32, 16)
        lane48 = lane * 48

        def slab_src(b, k):
            return rel.at[pl.ds((b * _NP + k * 128) * 48, 6144)]

        def load_ids(slot, m, cl):
            koff = slot * 6144 + cl * 48 + m * 4
            i0 = jnp.clip(plsc.load_gather(idx_v, [koff]), 0, 255)
            i1 = jnp.clip(plsc.load_gather(idx_v, [koff + 1]), 0, 255)
            i2 = jnp.clip(plsc.load_gather(idx_v, [koff + 2]), 0, 255)
            i3 = jnp.clip(plsc.load_gather(idx_v, [koff + 3]), 0, 255)
            return i0, i1, i2, i3

        @pl.when(wid < _NA)
        def _a_role():
            ta = wid
            b0 = ta // (_NA // _NB)
            pltpu.sync_copy(graph.at[b0], table_v.at[pl.ds(0, 256)])
            pltpu.sync_copy(discat, discat_v)
            pltpu.sync_copy(pinfo.at[b0], pinfo_v)
            r0 = lax.rem(ta * _AJOBS, _JPB)
            pltpu.sync_copy(slab_src(b0, lax.rem(r0, _NK)),
                            idx_v.at[pl.ds(0, 6144)])

            def unit(u, carry):
                i = u // 2
                half = lax.rem(u, 2)
                js = lax.rem(i, 2)
                r = lax.rem(ta * _AJOBS + i, _JPB)
                m = r // _NK
                k = lax.rem(r, _NK)

                @pl.when(jnp.logical_and(half == 0, i >= 1))
                def _ws():
                    pltpu.make_async_copy(
                        slab_src(b0, k),
                        idx_v.at[pl.ds(js * 6144, 6144)], sem2).wait()

                @pl.when(jnp.logical_and(half == 0, i + 1 < _AJOBS))
                def _ps():
                    r1 = lax.rem(ta * _AJOBS + i + 1, _JPB)
                    pltpu.async_copy(
                        slab_src(b0, lax.rem(r1, _NK)),
                        idx_v.at[pl.ds(lax.rem(i + 1, 2) * 6144, 6144)], sem2)

                @pl.when(jnp.logical_and(u >= 2, half == 0))
                def _wx():
                    pltpu.make_async_copy(
                        bufx_v,
                        outf.at[b0, 0, pl.ds(0, 144), pl.ds(0, 128)],
                        sem.at[0]).wait()

                @pl.when(jnp.logical_and(u >= 2, half == 1))
                def _wy():
                    pltpu.make_async_copy(
                        bufy_v,
                        outf.at[b0, 0, pl.ds(144, 152), pl.ds(0, 128)],
                        sem.at[1]).wait()

                def subchunk(c, cc):
                    cl = c * 16 + lane
                    i0, i1, i2, i3 = load_ids(js, m, cl)
                    isel = jnp.where(jnp.broadcast_to(m < 4, (16,)), i2, i3)
                    ssum = i0 + i1 + i2 + i3
                    mask_f = jnp.where(ssum > 0, 1.0, 0.0).astype(jnp.float32)

                    @pl.when(half == 0)
                    def _h0():
                        plsc.store_scatter(mask_v, [i * 128 + cl], ssum)

                        @plsc.parallel_loop(0, 128, unroll=4)
                        def _g0(cu):
                            cuv = jnp.broadcast_to(cu, (16,))
                            v = plsc.load_gather(table_v, [i0, cuv]) * mask_f
                            plsc.store_scatter(bufx_v, [cuv, cl], v)

                        @plsc.parallel_loop(0, 16, unroll=4)
                        def _g1(cu):
                            cuv = jnp.broadcast_to(cu, (16,))
                            v = plsc.load_gather(table_v, [isel, cuv]) * mask_f
                            plsc.store_scatter(bufx_v, [cuv + 128, cl], v)

                    @pl.when(half == 1)
                    def _h1():
                        a0 = plsc.load_gather(pinfo_v, [i0])
                        asel = plsc.load_gather(pinfo_v, [isel])
                        delta = a0 - asel
                        xeff = jnp.clip(
                            jnp.where(delta < 0, delta + 512, delta), 0, 511)
                        d = _bucket(xeff)
                        di = jnp.where(delta < 0, 10 - d, 10 + d)
                        di2 = _bucket(di) + 30

                        @plsc.parallel_loop(0, 112, unroll=4)
                        def _g2(cu):
                            cuv = jnp.broadcast_to(cu, (16,))
                            v = plsc.load_gather(
                                table_v, [isel, cuv + 16]) * mask_f
                            plsc.store_scatter(bufy_v, [cuv, cl], v)

                        @plsc.parallel_loop(0, 20, unroll=4)
                        def _g3(cu):
                            cuv = jnp.broadcast_to(cu, (16,))
                            e0 = plsc.load_gather(discat_v, [di, cuv]) * mask_f
                            plsc.store_scatter(bufy_v, [cuv + 112, cl], e0)
                            e1 = plsc.load_gather(discat_v, [di2, cuv]) * mask_f
                            plsc.store_scatter(bufy_v, [cuv + 132, cl], e1)

                    return cc
                lax.fori_loop(0, 8, subchunk, 0)

                @pl.when(half == 0)
                def _dx():
                    pltpu.async_copy(
                        bufx_v,
                        outf.at[b0, m, pl.ds(0, 144), pl.ds(k * 128, 128)],
                        sem.at[0])

                @pl.when(half == 1)
                def _dy():
                    pltpu.async_copy(
                        bufy_v,
                        outf.at[b0, m, pl.ds(144, 152), pl.ds(k * 128, 128)],
                        sem.at[1])
                return carry

            lax.fori_loop(0, 2 * _AJOBS, unit, 0)
            pltpu.make_async_copy(
                bufx_v, outf.at[b0, 0, pl.ds(0, 144), pl.ds(0, 128)],
                sem.at[0]).wait()
            pltpu.make_async_copy(
                bufy_v, outf.at[b0, 0, pl.ds(144, 152), pl.ds(0, 128)],
                sem.at[1]).wait()
            pltpu.sync_copy(mask_v, outm.at[ta])

        @pl.when(wid >= _NA)
        def _b_role():
            tb = wid - _NA
            b0 = (tb * _BJOBS) // _JPB
            pltpu.sync_copy(ctx.at[b0], table_v)
            pltpu.sync_copy(pinfo.at[b0], pinfo_v)
            r0 = lax.rem(tb * _BJOBS, _JPB)
            pltpu.sync_copy(slab_src(b0, lax.rem(r0, _NK)),
                            idx_v.at[pl.ds(0, 6144)])

            def unit(u, bcur):
                i = u // 2
                half = lax.rem(u, 2)
                js = lax.rem(i, 2)
                j = jnp.minimum(tb * _BJOBS + i, _JOBS - 1)
                bj = j // _JPB
                r = lax.rem(j, _JPB)
                m = r // _NK
                k = lax.rem(r, _NK)

                @pl.when(bj != bcur)
                def _reload():
                    pltpu.sync_copy(ctx.at[bj], table_v)
                    pltpu.sync_copy(pinfo.at[bj], pinfo_v)

                @pl.when(jnp.logical_and(half == 0, i >= 1))
                def _ws():
                    pltpu.make_async_copy(
                        slab_src(bj, k),
                        idx_v.at[pl.ds(js * 6144, 6144)], sem2).wait()

                @pl.when(jnp.logical_and(half == 0, i + 1 < _BJOBS))
                def _ps():
                    j1 = jnp.minimum(tb * _BJOBS + i + 1, _JOBS - 1)
                    r1 = lax.rem(j1, _JPB)
                    pltpu.async_copy(
                        slab_src(j1 // _JPB, lax.rem(r1, _NK)),
                        idx_v.at[pl.ds(lax.rem(i + 1, 2) * 6144, 6144)], sem2)

                @pl.when(jnp.logical_and(u >= 2, half == 0))
                def _wx():
                    pltpu.make_async_copy(
                        bufx_v.at[pl.ds(0, 128)],
                        outf.at[bj, 0, pl.ds(296, 128), pl.ds(0, 128)],
                        sem.at[0]).wait()

                @pl.when(jnp.logical_and(u >= 2, half == 1))
                def _wy():
                    pltpu.make_async_copy(
                        bufy_v.at[pl.ds(0, 128)],
                        outf.at[bj, 0, pl.ds(424, 128), pl.ds(0, 128)],
                        sem.at[1]).wait()

                def subchunk(c, cc):
                    cl = c * 16 + lane
                    i0, i1, i2, i3 = load_ids(js, m, cl)
                    ssum = i0 + i1 + i2 + i3
                    mask_f = jnp.where(ssum > 0, 1.0, 0.0).astype(jnp.float32)
                    wlog = jnp.where(jnp.broadcast_to(m >= 8, (16,)),
                                     1.0, 0.0).astype(jnp.float32)
                    a0 = jnp.clip(plsc.load_gather(pinfo_v, [i0]), 0, 499)
                    a1 = jnp.clip(plsc.load_gather(pinfo_v, [i1]), 0, 499)
                    a2 = jnp.clip(plsc.load_gather(pinfo_v, [i2]), 0, 499)
                    a3 = jnp.clip(plsc.load_gather(pinfo_v, [i3]), 0, 499)

                    @pl.when(half == 0)
                    def _h0():
                        @plsc.parallel_loop(0, 128, unroll=4)
                        def _c0(cu):
                            cuv = jnp.broadcast_to(cu, (16,))
                            u0 = plsc.load_gather(table_v, [a0, cuv])
                            u1 = plsc.load_gather(table_v, [a1, cuv])
                            plsc.store_scatter(bufx_v, [cuv, cl],
                                               (u0 + wlog * u1) * mask_f)

                    @pl.when(half == 1)
                    def _h1():
                        @plsc.parallel_loop(0, 128, unroll=4)
                        def _c1(cu):
                            cuv = jnp.broadcast_to(cu, (16,))
                            u2 = plsc.load_gather(table_v, [a2, cuv])
                            u3 = plsc.load_gather(table_v, [a3, cuv])
                            plsc.store_scatter(bufy_v, [cuv, cl],
                                               (u2 + wlog * u3) * mask_f)

                    return cc
                lax.fori_loop(0, 8, subchunk, 0)

                @pl.when(half == 0)
                def _dx():
                    pltpu.async_copy(
                        bufx_v.at[pl.ds(0, 128)],
                        outf.at[bj, m, pl.ds(296, 128), pl.ds(k * 128, 128)],
                        sem.at[0])

                @pl.when(half == 1)
                def _dy():
                    pltpu.async_copy(
                        bufy_v.at[pl.ds(0, 128)],
                        outf.at[bj, m, pl.ds(424, 128), pl.ds(k * 128, 128)],
                        sem.at[1])
                return bj

            lax.fori_loop(0, 2 * _BJOBS, unit, b0)
            pltpu.make_async_copy(
                bufx_v.at[pl.ds(0, 128)],
                outf.at[0, 0, pl.ds(296, 128), pl.ds(0, 128)],
                sem.at[0]).wait()
            pltpu.make_async_copy(
                bufy_v.at[pl.ds(0, 128)],
                outf.at[0, 0, pl.ds(424, 128), pl.ds(0, 128)],
                sem.at[1]).wait()

    return sc_kernel


_sc_call = _make_sc_call()


@jax.jit
def kernel(relation_path, path_info, graph_feature, context_feature,
           dis_embed, dis_sent_embed):
    rel1 = jnp.concatenate([
        relation_path.astype(jnp.int32).reshape(_NB * _NP * _NM * 4),
        jnp.zeros((_RPAD,), jnp.int32)])
    pinfo0 = path_info.astype(jnp.int32)[:, :, 0]
    gf = graph_feature.astype(jnp.float32)
    cf = context_feature.astype(jnp.float32)[:, :500, :]
    discat = jnp.concatenate(
        [dis_embed.astype(jnp.float32), dis_sent_embed.astype(jnp.float32)],
        axis=0)
    outf, outm = _sc_call(rel1, pinfo0, gf, cf, discat)
    path_fea = jnp.transpose(outf, (0, 3, 1, 2))[:, :_NP]
    mm = outm.reshape(_JOBS, 128).reshape(_NB, _NM, _NK * 128)[:, :, :_NP]
    mask = jnp.transpose(mm > 0, (0, 2, 1))
    return (path_fea, mask)
